# baseline (speedup 1.0000x reference)
import functools

import jax
import jax.numpy as jnp
from jax import lax
from jax.experimental import pallas as pl
from jax.experimental.pallas import tpu as pltpu
from jax.experimental.pallas import tpu_sc as plsc

_B = 64


def kernel(x, patch_mask, mask_token):
    N, L, H, W, D = x.shape
    rows = N * L * H * W
    xf = x.reshape(rows, D)
    mf = patch_mask.reshape(rows).astype(jnp.int32)
    tok = mask_token.reshape(D)

    info = plsc.get_sparse_core_info()
    nw = info.num_cores * info.num_subcores
    rpw = rows // nw
    nblk = rpw // _B
    ngrp = D // 16

    mesh = plsc.VectorSubcoreMesh(core_axis_name="c", subcore_axis_name="s")

    @functools.partial(
        pl.kernel,
        mesh=mesh,
        out_type=jax.ShapeDtypeStruct((rows, D), jnp.float32),
        scratch_types=[
            pltpu.VMEM((_B, D), jnp.float32),
            pltpu.VMEM((_B,), jnp.int32),
            pltpu.VMEM((D,), jnp.float32),
        ],
    )
    def k(x_hbm, m_hbm, t_hbm, out_hbm, buf, mb, tokv):
        wid = lax.axis_index("s") * info.num_cores + lax.axis_index("c")
        base = wid * rpw

        pltpu.sync_copy(t_hbm, tokv)

        def blk_body(i, carry):
            r0 = base + i * _B
            pltpu.sync_copy(x_hbm.at[pl.ds(r0, _B)], buf)
            pltpu.sync_copy(m_hbm.at[pl.ds(r0, _B)], mb)

            def grp_body(g, c2):
                mv = mb[pl.ds(g * 16, 16)]
                for j in range(16):
                    @pl.when(mv[j] != 0)
                    def _():
                        r = g * 16 + j
                        for c in range(ngrp):
                            buf[r, pl.ds(c * 16, 16)] = tokv[pl.ds(c * 16, 16)]
                return c2

            lax.fori_loop(0, _B // 16, grp_body, 0)
            pltpu.sync_copy(buf, out_hbm.at[pl.ds(r0, _B)])
            return carry

        lax.fori_loop(0, nblk, blk_body, 0)

    out = k(xf, mf, tok)
    return (out.reshape(x.shape), patch_mask)


# speedup vs baseline: 1.7425x; 1.7425x over previous
import functools

import jax
import jax.numpy as jnp
from jax import lax
from jax.experimental import pallas as pl
from jax.experimental.pallas import tpu as pltpu
from jax.experimental.pallas import tpu_sc as plsc

_B = 32
_NBUF = 4


def kernel(x, patch_mask, mask_token):
    N, L, H, W, D = x.shape
    rows = N * L * H * W
    xf = x.reshape(rows, D)
    mf = patch_mask.reshape(rows).astype(jnp.int32)
    tok = mask_token.reshape(D)

    info = plsc.get_sparse_core_info()
    nw = info.num_cores * info.num_subcores
    rpw = rows // nw
    nblk = rpw // _B
    nsuper = nblk // _NBUF
    ngrp = D // 16

    mesh = plsc.VectorSubcoreMesh(core_axis_name="c", subcore_axis_name="s")

    scratch = (
        [pltpu.VMEM((_B, D), jnp.float32) for _ in range(_NBUF)]
        + [pltpu.VMEM((_B + 16,), jnp.int32) for _ in range(_NBUF)]
        + [pltpu.VMEM((D,), jnp.float32)]
        + [pltpu.SemaphoreType.DMA for _ in range(3 * _NBUF)]
    )

    @functools.partial(
        pl.kernel,
        mesh=mesh,
        out_type=jax.ShapeDtypeStruct((rows, D), jnp.float32),
        scratch_types=scratch,
    )
    def k(x_hbm, m_hbm, t_hbm, out_hbm, *refs):
        bufs = refs[:_NBUF]
        mbs = refs[_NBUF:2 * _NBUF]
        tokv = refs[2 * _NBUF]
        gxs = refs[2 * _NBUF + 1:2 * _NBUF + 1 + _NBUF]
        gms = refs[2 * _NBUF + 1 + _NBUF:2 * _NBUF + 1 + 2 * _NBUF]
        scs = refs[2 * _NBUF + 1 + 2 * _NBUF:]

        wid = lax.axis_index("s") * info.num_cores + lax.axis_index("c")
        base = wid * rpw

        pltpu.sync_copy(t_hbm, tokv)

        def gather(blk, b):
            r0 = base + blk * _B
            pltpu.make_async_copy(x_hbm.at[pl.ds(r0, _B)], bufs[b], gxs[b]).start()
            pltpu.make_async_copy(m_hbm.at[pl.ds(r0, _B)], mbs[b].at[pl.ds(0, _B)],
                                  gms[b]).start()

        def gather_wait(blk, b):
            r0 = base + blk * _B
            pltpu.make_async_copy(x_hbm.at[pl.ds(r0, _B)], bufs[b], gxs[b]).wait()
            pltpu.make_async_copy(m_hbm.at[pl.ds(r0, _B)], mbs[b].at[pl.ds(0, _B)],
                                  gms[b]).wait()

        def scatter(blk, b):
            r0 = base + blk * _B
            pltpu.make_async_copy(bufs[b], out_hbm.at[pl.ds(r0, _B)], scs[b]).start()

        def scatter_wait(b):
            pltpu.make_async_copy(bufs[b], out_hbm.at[pl.ds(base, _B)], scs[b]).wait()

        def process(b):
            buf, mb = bufs[b], mbs[b]

            def row_body(r, c2):
                mv = mb[pl.ds(r, 16)]

                @pl.when(mv[0] != 0)
                def _():
                    for c in range(ngrp):
                        buf[r, pl.ds(c * 16, 16)] = tokv[pl.ds(c * 16, 16)]
                return c2

            lax.fori_loop(0, _B, row_body, 0)

        def super_body(i, carry):
            for b in range(_NBUF):
                @pl.when(i > 0)
                def _(b=b):
                    scatter_wait(b)
                gather(i * _NBUF + b, b)
            for b in range(_NBUF):
                gather_wait(i * _NBUF + b, b)
                process(b)
                scatter(i * _NBUF + b, b)
            return carry

        lax.fori_loop(0, nsuper, super_body, 0)
        for b in range(_NBUF):
            scatter_wait(b)

    out = k(xf, mf, tok)
    return (out.reshape(x.shape), patch_mask)


# speedup vs baseline: 4.4451x; 2.5511x over previous
import jax
import jax.numpy as jnp
from jax.experimental import pallas as pl
from jax.experimental.pallas import tpu as pltpu

_ROWS_PER_BLOCK = 4608


def _select_body(m_ref, t_ref, x_ref, o_ref):
    i = pl.program_id(0)
    m = m_ref[i, :].astype(jnp.int32)
    tok = t_ref[0, :]
    o_ref[:, :] = jnp.where(m[:, None] != 0, tok[None, :], x_ref[:, :])


def kernel(x, patch_mask, mask_token):
    N, L, H, W, D = x.shape
    rows = N * L * H * W
    xf = x.reshape(rows, D)
    nblk = rows // _ROWS_PER_BLOCK
    mf = patch_mask.reshape(nblk, _ROWS_PER_BLOCK)

    out = pl.pallas_call(
        _select_body,
        grid=(nblk,),
        in_specs=[
            pl.BlockSpec((nblk, _ROWS_PER_BLOCK), lambda i: (0, 0)),
            pl.BlockSpec((1, D), lambda i: (0, 0)),
            pl.BlockSpec((_ROWS_PER_BLOCK, D), lambda i: (i, 0)),
        ],
        out_specs=pl.BlockSpec((_ROWS_PER_BLOCK, D), lambda i: (i, 0)),
        out_shape=jax.ShapeDtypeStruct((rows, D), x.dtype),
        compiler_params=pltpu.CompilerParams(
            dimension_semantics=("parallel",),
        ),
    )(mf, mask_token, xf)

    return (out.reshape(x.shape), patch_mask)


# speedup vs baseline: 4.4476x; 1.0006x over previous
import jax
import jax.numpy as jnp
from jax.experimental import pallas as pl
from jax.experimental.pallas import tpu as pltpu

_ROWS_PER_BLOCK = 4608


def _select_body(m_ref, t_ref, x_ref, o_ref):
    i = pl.program_id(0)
    m = m_ref[i, :].astype(jnp.int32)
    tok = t_ref[0, :]
    o_ref[:, :] = jnp.where(m[:, None] != 0, tok[None, :], x_ref[:, :])


def kernel(x, patch_mask, mask_token):
    N, L, H, W, D = x.shape
    rows = N * L * H * W
    xf = x.reshape(rows, D)
    nblk = rows // _ROWS_PER_BLOCK
    mf = patch_mask.reshape(nblk, _ROWS_PER_BLOCK)

    out = pl.pallas_call(
        _select_body,
        grid=(nblk,),
        in_specs=[
            pl.BlockSpec((nblk, _ROWS_PER_BLOCK), lambda i: (0, 0)),
            pl.BlockSpec((1, D), lambda i: (0, 0)),
            pl.BlockSpec((_ROWS_PER_BLOCK, D), lambda i: (i, 0)),
        ],
        out_specs=pl.BlockSpec((_ROWS_PER_BLOCK, D), lambda i: (i, 0)),
        out_shape=jax.ShapeDtypeStruct((rows, D), x.dtype),
        compiler_params=pltpu.CompilerParams(
            dimension_semantics=("arbitrary",),
        ),
    )(mf, mask_token, xf)

    return (out.reshape(x.shape), patch_mask)
